# baseline (speedup 1.0000x reference)
import functools

import jax
import jax.numpy as jnp
from jax.experimental import pallas as pl
from jax.experimental.pallas import tpu as pltpu


def _router_kernel(x_ref, w_ref, b_ref, out_ref, cnt_ref):
    acc = jnp.dot(x_ref[...], w_ref[...], preferred_element_type=jnp.float32)
    logits = jnp.maximum(acc + b_ref[...], 0.0)
    out_ref[...] = logits
    nz = jnp.sum((logits > 0.0).astype(jnp.float32))
    cnt_ref[...] = jnp.full(cnt_ref.shape, nz, dtype=jnp.float32)


@functools.partial(jax.jit, static_argnames=("block_t",))
def _run(x, W, b, block_t):
    n_tokens, d_model = x.shape
    n_experts = W.shape[1]
    n_tiles = n_tokens // block_t
    b2 = b.reshape(1, n_experts)

    logits, counts = pl.pallas_call(
        _router_kernel,
        grid=(n_tiles,),
        in_specs=[
            pl.BlockSpec((block_t, d_model), lambda i: (i, 0)),
            pl.BlockSpec((d_model, n_experts), lambda i: (0, 0)),
            pl.BlockSpec((1, n_experts), lambda i: (0, 0)),
        ],
        out_specs=[
            pl.BlockSpec((block_t, n_experts), lambda i: (i, 0)),
            pl.BlockSpec((1, 1, 128), lambda i: (i, 0, 0)),
        ],
        out_shape=[
            jax.ShapeDtypeStruct((n_tokens, n_experts), jnp.float32),
            jax.ShapeDtypeStruct((n_tiles, 1, 128), jnp.float32),
        ],
        compiler_params=pltpu.CompilerParams(
            dimension_semantics=("parallel",)
        ),
    )(x, W, b2)

    density = jnp.sum(counts[:, 0, 0]) / (n_tokens * n_experts)
    return logits, density.astype(jnp.float32)


def kernel(x, W, b):
    return _run(x, W, b, 256)


# speedup vs baseline: 1.2229x; 1.2229x over previous
import functools

import jax
import jax.numpy as jnp
from jax.experimental import pallas as pl
from jax.experimental.pallas import tpu as pltpu

NBUF = 4
CHUNK = 512


def _router_kernel(n_chunks, x_hbm, w_ref, b_ref, out_ref, cnt_ref, buf, sems):
    def start_copy(c):
        slot = jax.lax.rem(c, NBUF)
        pltpu.make_async_copy(
            x_hbm.at[pl.ds(c * CHUNK, CHUNK), :],
            buf.at[slot],
            sems.at[slot],
        ).start()

    for c in range(min(NBUF, n_chunks)):
        start_copy(c)

    def step(c, count):
        slot = jax.lax.rem(c, NBUF)
        pltpu.make_async_copy(
            x_hbm.at[pl.ds(c * CHUNK, CHUNK), :],
            buf.at[slot],
            sems.at[slot],
        ).wait()
        acc = jnp.dot(buf[slot], w_ref[...], preferred_element_type=jnp.float32)
        logits = jnp.maximum(acc + b_ref[...], 0.0)
        out_ref[pl.ds(c * CHUNK, CHUNK), :] = logits
        count = count + jnp.sum((logits > 0.0).astype(jnp.float32))

        nxt = c + NBUF

        @pl.when(nxt < n_chunks)
        def _():
            start_copy(nxt)

        return count

    count = jax.lax.fori_loop(0, n_chunks, step, jnp.float32(0.0))
    cnt_ref[...] = jnp.full(cnt_ref.shape, count, dtype=jnp.float32)


@jax.jit
def _run(x, W, b):
    n_tokens, d_model = x.shape
    n_experts = W.shape[1]
    n_chunks = n_tokens // CHUNK
    b2 = b.reshape(1, n_experts)

    logits, counts = pl.pallas_call(
        functools.partial(_router_kernel, n_chunks),
        in_specs=[
            pl.BlockSpec(memory_space=pl.ANY),
            pl.BlockSpec(memory_space=pltpu.VMEM),
            pl.BlockSpec(memory_space=pltpu.VMEM),
        ],
        out_specs=[
            pl.BlockSpec(memory_space=pltpu.VMEM),
            pl.BlockSpec(memory_space=pltpu.VMEM),
        ],
        out_shape=[
            jax.ShapeDtypeStruct((n_tokens, n_experts), jnp.float32),
            jax.ShapeDtypeStruct((8, 128), jnp.float32),
        ],
        scratch_shapes=[
            pltpu.VMEM((NBUF, CHUNK, d_model), jnp.float32),
            pltpu.SemaphoreType.DMA((NBUF,)),
        ],
        compiler_params=pltpu.CompilerParams(
            vmem_limit_bytes=100 * 1024 * 1024,
        ),
    )(x, W, b2)

    density = counts[0, 0] / (n_tokens * n_experts)
    return logits, density.astype(jnp.float32)


def kernel(x, W, b):
    return _run(x, W, b)


# speedup vs baseline: 1.2435x; 1.0168x over previous
import functools

import jax
import jax.numpy as jnp
from jax.experimental import pallas as pl
from jax.experimental.pallas import tpu as pltpu


def _router_kernel(x_ref, w_ref, b_ref, out_ref, cnt_ref):
    x16 = x_ref[...].astype(jnp.bfloat16)
    acc = jnp.dot(x16, w_ref[...], preferred_element_type=jnp.float32)
    logits = jnp.maximum(acc + b_ref[...], 0.0)
    out_ref[...] = logits
    nz = jnp.sum((logits > 0.0).astype(jnp.float32))
    cnt_ref[...] = jnp.full(cnt_ref.shape, nz, dtype=jnp.float32)


@functools.partial(jax.jit, static_argnames=("block_t",))
def _run(x, W, b, block_t):
    n_tokens, d_model = x.shape
    n_experts = W.shape[1]
    n_tiles = n_tokens // block_t
    b2 = b.reshape(1, n_experts)
    Wb = W.astype(jnp.bfloat16)

    logits, counts = pl.pallas_call(
        _router_kernel,
        grid=(n_tiles,),
        in_specs=[
            pl.BlockSpec((block_t, d_model), lambda i: (i, 0)),
            pl.BlockSpec((d_model, n_experts), lambda i: (0, 0)),
            pl.BlockSpec((1, n_experts), lambda i: (0, 0)),
        ],
        out_specs=[
            pl.BlockSpec((block_t, n_experts), lambda i: (i, 0)),
            pl.BlockSpec((1, 1, 128), lambda i: (i, 0, 0)),
        ],
        out_shape=[
            jax.ShapeDtypeStruct((n_tokens, n_experts), jnp.float32),
            jax.ShapeDtypeStruct((n_tiles, 1, 128), jnp.float32),
        ],
        compiler_params=pltpu.CompilerParams(
            dimension_semantics=("arbitrary",),
            vmem_limit_bytes=110 * 1024 * 1024,
            fuse_transposed_lhs_in_matmul=True,
        ),
    )(x, Wb, b2)

    density = jnp.sum(counts[:, 0, 0]) / (n_tokens * n_experts)
    return logits, density.astype(jnp.float32)


def kernel(x, W, b):
    return _run(x, W, b, 1024)
